# baseline (speedup 1.0000x reference)
import functools

import jax
import jax.numpy as jnp
from jax.experimental import pallas as pl
from jax.experimental.pallas import tpu as pltpu

_NBUF = 4


def _body9(attrs_ref, vec_ref, out_hbm, vt_s, bufs, *sems):
    ai = pl.program_id(0)
    na = pl.num_programs(0)
    n, f = vec_ref.shape
    bi = n // _NBUF

    @pl.when(ai == 0)
    def _():
        for c in range(n // 128):
            vt_s[:, c * 128:(c + 1) * 128] = vec_ref[c * 128:(c + 1) * 128, :].T

    attr = attrs_ref[ai]
    col = vt_s[pl.ds(attr, 1), :]
    for s in range(_NBUF):
        @pl.when(ai > 0)
        def _():
            pltpu.make_async_copy(
                bufs.at[s], out_hbm.at[ai, pl.ds(s * bi, bi), :], sems[s]
            ).wait()

        rows = vt_s[pl.ds(attr, 1), pl.ds(s * bi, bi)]
        bufs[s, :, :] = jnp.abs(rows[0][:, None] - col)
        pltpu.make_async_copy(
            bufs.at[s], out_hbm.at[ai, pl.ds(s * bi, bi), :], sems[s]
        ).start()

    @pl.when(ai == na - 1)
    def _():
        for s in range(_NBUF):
            pltpu.make_async_copy(
                bufs.at[s], out_hbm.at[ai, pl.ds(s * bi, bi), :], sems[s]
            ).wait()


def kernel(vectors, attributes):
    n, f = vectors.shape
    a = attributes.shape[0]
    bi = n // _NBUF

    out = pl.pallas_call(
        _body9,
        grid=(a,),
        in_specs=[
            pl.BlockSpec(memory_space=pltpu.SMEM),
            pl.BlockSpec((n, f), lambda ai: (0, 0)),
        ],
        out_specs=pl.BlockSpec(memory_space=pl.ANY),
        out_shape=jax.ShapeDtypeStruct((a, n, n), jnp.float32),
        scratch_shapes=[
            pltpu.VMEM((f, n), jnp.float32),
            pltpu.VMEM((_NBUF, bi, n), jnp.float32),
        ] + [pltpu.SemaphoreType.DMA] * _NBUF,
        compiler_params=pltpu.CompilerParams(
            dimension_semantics=("arbitrary",),
        ),
    )(attributes.astype(jnp.int32), vectors)
    return out


# speedup vs baseline: 1.0133x; 1.0133x over previous
import functools

import jax
import jax.numpy as jnp
from jax.experimental import pallas as pl
from jax.experimental.pallas import tpu as pltpu


def _body(attrs_ref, vec_ref, out_ref, vt_s, *, block_i: int):
    ai = pl.program_id(0)
    i = pl.program_id(1)

    @pl.when((ai == 0) & (i == 0))
    def _():
        n, f = vec_ref.shape
        for c in range(n // 128):
            vt_s[:, c * 128:(c + 1) * 128] = vec_ref[c * 128:(c + 1) * 128, :].T

    attr = attrs_ref[ai]
    col = vt_s[pl.ds(attr, 1), :]
    rows = vt_s[pl.ds(attr, 1), pl.ds(i * block_i, block_i)]
    out_ref[0, :, :] = jnp.abs(rows[0][:, None] - col)


def kernel(vectors, attributes):
    n, f = vectors.shape
    a = attributes.shape[0]
    block_i = 512
    grid = (a, n // block_i)

    body = functools.partial(_body, block_i=block_i)
    out = pl.pallas_call(
        body,
        grid=grid,
        in_specs=[
            pl.BlockSpec(memory_space=pltpu.SMEM),
            pl.BlockSpec((n, f), lambda ai, i: (0, 0)),
        ],
        out_specs=pl.BlockSpec((1, block_i, n), lambda ai, i: (ai, i, 0)),
        out_shape=jax.ShapeDtypeStruct((a, n, n), jnp.float32),
        scratch_shapes=[pltpu.VMEM((f, n), jnp.float32)],
        compiler_params=pltpu.CompilerParams(
            dimension_semantics=("arbitrary", "arbitrary"),
        ),
    )(attributes.astype(jnp.int32), vectors)
    return out


# speedup vs baseline: 1.0252x; 1.0117x over previous
import functools

import jax
import jax.numpy as jnp
from jax.experimental import pallas as pl
from jax.experimental.pallas import tpu as pltpu

_NCHUNK = 8


def _body(attrs_ref, vec_hbm, out_ref, vec_v, vt_s, *sems, block_i: int):
    ai = pl.program_id(0)
    i = pl.program_id(1)

    @pl.when((ai == 0) & (i == 0))
    def _():
        n, f = vec_hbm.shape
        rc = n // _NCHUNK
        copies = []
        for c in range(_NCHUNK):
            cp = pltpu.make_async_copy(
                vec_hbm.at[pl.ds(c * rc, rc), :], vec_v.at[pl.ds(c * rc, rc), :],
                sems[c],
            )
            cp.start()
            copies.append(cp)
        for c in range(_NCHUNK):
            copies[c].wait()
            for cc in range(rc // 128):
                r0 = c * rc + cc * 128
                vt_s[:, r0:r0 + 128] = vec_v[r0:r0 + 128, :].T

    attr = attrs_ref[ai]
    col = vt_s[pl.ds(attr, 1), :]
    rows = vt_s[pl.ds(attr, 1), pl.ds(i * block_i, block_i)]
    out_ref[0, :, :] = jnp.abs(rows[0][:, None] - col)


def kernel(vectors, attributes):
    n, f = vectors.shape
    a = attributes.shape[0]
    block_i = 512
    grid = (a, n // block_i)

    body = functools.partial(_body, block_i=block_i)
    out = pl.pallas_call(
        body,
        grid=grid,
        in_specs=[
            pl.BlockSpec(memory_space=pltpu.SMEM),
            pl.BlockSpec(memory_space=pl.ANY),
        ],
        out_specs=pl.BlockSpec((1, block_i, n), lambda ai, i: (ai, i, 0)),
        out_shape=jax.ShapeDtypeStruct((a, n, n), jnp.float32),
        scratch_shapes=[
            pltpu.VMEM((n, f), jnp.float32),
            pltpu.VMEM((f, n), jnp.float32),
        ] + [pltpu.SemaphoreType.DMA] * _NCHUNK,
        compiler_params=pltpu.CompilerParams(
            dimension_semantics=("arbitrary", "arbitrary"),
        ),
    )(attributes.astype(jnp.int32), vectors)
    return out
